# baseline (speedup 1.0000x reference)
import jax
import jax.numpy as jnp
from jax import lax
from jax.experimental import pallas as pl
from jax.experimental.pallas import tpu as pltpu

N = 4096
D_IN = 128
D_OUT = 128
B = 2048
TAIL = 512
HEAD = B - TAIL
UNROLL = 64


def _body(
    idx_sref,
    x_any,
    w_any,
    b_any,
    ppr_any,
    out_ref,
    x_v,
    w_v,
    b_v,
    enc_ref,
    buf_ref,
    sem,
    sem_s,
):
    def issue(lo, n, part):
        def one(r, _):
            row = idx_sref[lo + r]
            pltpu.make_async_copy(
                ppr_any.at[row], buf_ref.at[lo + r], sem.at[part]
            ).start()
            return 0

        lax.fori_loop(0, n, one, 0, unroll=UNROLL)

    issue(0, HEAD, 0)
    cx = pltpu.make_async_copy(x_any, x_v, sem_s)
    cw = pltpu.make_async_copy(w_any, w_v, sem_s)
    cb = pltpu.make_async_copy(b_any, b_v, sem_s)
    cx.start()
    cw.start()
    cb.start()
    issue(HEAD, TAIL, 1)
    cx.wait()
    cw.wait()
    cb.wait()
    enc_ref[...] = (
        jnp.dot(x_v[...], w_v[...], preferred_element_type=jnp.float32) + b_v[...]
    )

    pltpu.make_async_copy(
        ppr_any.at[pl.ds(0, HEAD)], buf_ref.at[pl.ds(0, HEAD)], sem.at[0]
    ).wait()
    out_ref[pl.ds(0, HEAD), :] = jnp.dot(
        buf_ref[pl.ds(0, HEAD), :], enc_ref[...], preferred_element_type=jnp.float32
    )

    pltpu.make_async_copy(
        ppr_any.at[pl.ds(0, TAIL)], buf_ref.at[pl.ds(HEAD, TAIL)], sem.at[1]
    ).wait()
    out_ref[pl.ds(HEAD, TAIL), :] = jnp.dot(
        buf_ref[pl.ds(HEAD, TAIL), :],
        enc_ref[...],
        preferred_element_type=jnp.float32,
    )


def kernel(X, idx, ppr, W, b):
    grid_spec = pltpu.PrefetchScalarGridSpec(
        num_scalar_prefetch=1,
        grid=(1,),
        in_specs=[
            pl.BlockSpec(memory_space=pl.ANY),
            pl.BlockSpec(memory_space=pl.ANY),
            pl.BlockSpec(memory_space=pl.ANY),
            pl.BlockSpec(memory_space=pl.ANY),
        ],
        out_specs=pl.BlockSpec((B, D_OUT), lambda i, idx_ref: (0, 0)),
        scratch_shapes=[
            pltpu.VMEM((N, D_IN), jnp.float32),
            pltpu.VMEM((D_IN, D_OUT), jnp.float32),
            pltpu.VMEM((1, D_OUT), jnp.float32),
            pltpu.VMEM((N, D_OUT), jnp.float32),
            pltpu.VMEM((B, N), jnp.float32),
            pltpu.SemaphoreType.DMA((2,)),
            pltpu.SemaphoreType.DMA,
        ],
    )
    return pl.pallas_call(
        _body,
        grid_spec=grid_spec,
        out_shape=jax.ShapeDtypeStruct((B, D_OUT), jnp.float32),
    )(idx.astype(jnp.int32), X, W, b.reshape(1, D_OUT), ppr)


# speedup vs baseline: 1.1225x; 1.1225x over previous
import jax
import jax.numpy as jnp
from jax import lax
from jax.experimental import pallas as pl
from jax.experimental.pallas import tpu as pltpu

N = 4096
D_IN = 128
D_OUT = 128
B = 2048
BQ = 1024
UNROLL = 64


def _body(
    idx_sref,
    x_any,
    w_any,
    b_any,
    ppr_any,
    out_ref,
    x_v,
    w_v,
    b_v,
    enc_ref,
    buf_ref,
    sem,
    sem_s,
):
    i = pl.program_id(0)
    nsteps = pl.num_programs(0)

    def issue(block, slot):
        def one(r, _):
            row = idx_sref[block * BQ + r]
            pltpu.make_async_copy(
                ppr_any.at[row], buf_ref.at[slot, r], sem.at[slot]
            ).start()
            return 0

        lax.fori_loop(0, BQ, one, 0, unroll=UNROLL)

    @pl.when(i == 0)
    def _():
        issue(0, 0)
        cx = pltpu.make_async_copy(x_any, x_v, sem_s)
        cw = pltpu.make_async_copy(w_any, w_v, sem_s)
        cb = pltpu.make_async_copy(b_any, b_v, sem_s)
        cx.start()
        cw.start()
        cb.start()
        issue(1, 1)
        cx.wait()
        cw.wait()
        cb.wait()
        enc_ref[...] = (
            jnp.dot(x_v[...], w_v[...], preferred_element_type=jnp.float32)
            + b_v[...]
        )

    @pl.when((i > 0) & (i + 1 < nsteps))
    def _():
        issue(i + 1, (i + 1) % 2)

    slot = i % 2
    pltpu.make_async_copy(
        ppr_any.at[pl.ds(0, BQ)], buf_ref.at[slot], sem.at[slot]
    ).wait()
    out_ref[...] = jnp.dot(
        buf_ref[slot], enc_ref[...], preferred_element_type=jnp.float32
    )


def kernel(X, idx, ppr, W, b):
    grid_spec = pltpu.PrefetchScalarGridSpec(
        num_scalar_prefetch=1,
        grid=(B // BQ,),
        in_specs=[
            pl.BlockSpec(memory_space=pl.ANY),
            pl.BlockSpec(memory_space=pl.ANY),
            pl.BlockSpec(memory_space=pl.ANY),
            pl.BlockSpec(memory_space=pl.ANY),
        ],
        out_specs=pl.BlockSpec((BQ, D_OUT), lambda i, idx_ref: (i, 0)),
        scratch_shapes=[
            pltpu.VMEM((N, D_IN), jnp.float32),
            pltpu.VMEM((D_IN, D_OUT), jnp.float32),
            pltpu.VMEM((1, D_OUT), jnp.float32),
            pltpu.VMEM((N, D_OUT), jnp.float32),
            pltpu.VMEM((2, BQ, N), jnp.float32),
            pltpu.SemaphoreType.DMA((2,)),
            pltpu.SemaphoreType.DMA,
        ],
    )
    return pl.pallas_call(
        _body,
        grid_spec=grid_spec,
        out_shape=jax.ShapeDtypeStruct((B, D_OUT), jnp.float32),
    )(idx.astype(jnp.int32), X, W, b.reshape(1, D_OUT), ppr)
